# baseline (speedup 1.0000x reference)
import jax
import jax.numpy as jnp
from jax.experimental import pallas as pl
from jax.experimental.pallas import tpu as pltpu


def kernel(predictions, image_grids, target_boxes_grid):
    B, H, W = predictions.shape

    def body(x_ref, g_ref, o_ref):
        x = x_ref[...]
        m = g_ref[0, 0].astype(jnp.float32)
        s_pm = jnp.sum(x * m, axis=(1, 2))
        cnt = jnp.sum(m, axis=(1, 2))
        s_p = jnp.sum(x, axis=(1, 2))
        mean_t = s_pm / cnt
        lt = -jnp.log(mean_t)
        mean_b = (s_p - s_pm) / (H * W - cnt)
        lb = jnp.nan_to_num(-jnp.log(1.0 - mean_b))
        o_ref[...] = ((jnp.sum(lb) + jnp.sum(lt)) / B).reshape(1, 1)

    out = pl.pallas_call(
        body,
        grid=(1,),
        in_specs=[
            pl.BlockSpec((B, H, W), lambda i: (0, 0, 0)),
            pl.BlockSpec((1, 1, B, H, W), lambda i: (H, W, 0, 0, 0)),
        ],
        out_specs=pl.BlockSpec((1, 1), lambda i: (0, 0)),
        out_shape=jax.ShapeDtypeStruct((1, 1), jnp.float32),
    )(predictions, image_grids)
    return out[0, 0]


# speedup vs baseline: 171.3231x; 171.3231x over previous
import jax
import jax.numpy as jnp
from jax.experimental import pallas as pl
from jax.experimental.pallas import tpu as pltpu


def kernel(predictions, image_grids, target_boxes_grid):
    B, H, W = predictions.shape
    pred_t = jnp.transpose(predictions, (1, 2, 0))
    grids_t = jnp.transpose(image_grids, (0, 3, 4, 1, 2))

    def body(x_ref, g_hbm, o_ref, gbuf, sem):
        cp = pltpu.make_async_copy(
            g_hbm.at[H, :, :, pl.ds(W, 1), :], gbuf, sem
        )
        cp.start()
        cp.wait()
        x = x_ref[...]
        m = gbuf[...].astype(jnp.float32)
        xm = x[:, :, None, :] * m
        s_pm = jnp.sum(xm, axis=(0, 1, 2))
        cnt = jnp.sum(m, axis=(0, 1, 2))
        s_p = jnp.sum(x, axis=(0, 1))
        mean_t = s_pm / cnt
        lt = -jnp.log(mean_t)
        mean_b = (s_p - s_pm) / (H * W - cnt)
        lb = jnp.nan_to_num(-jnp.log(1.0 - mean_b))
        o_ref[...] = ((jnp.sum(lb) + jnp.sum(lt)) / B).reshape(1, 1)

    out = pl.pallas_call(
        body,
        grid=(1,),
        in_specs=[
            pl.BlockSpec((H, W, B), lambda i: (0, 0, 0)),
            pl.BlockSpec(memory_space=pl.ANY),
        ],
        out_specs=pl.BlockSpec((1, 1), lambda i: (0, 0)),
        out_shape=jax.ShapeDtypeStruct((1, 1), jnp.float32),
        scratch_shapes=[
            pltpu.VMEM((H, W, 1, B), jnp.int32),
            pltpu.SemaphoreType.DMA,
        ],
    )(pred_t, grids_t)
    return out[0, 0]
